# Initial kernel scaffold; baseline (speedup 1.0000x reference)
import jax
import jax.numpy as jnp
from jax.experimental import pallas as pl


def kernel(z, weight):
    raise NotImplementedError("write your pallas kernel here")


# speedup vs baseline: 1.0058x; 1.0058x over previous
import functools

import jax
import jax.numpy as jnp
from jax import lax
from jax.experimental import pallas as pl
from jax.experimental.pallas import tpu as pltpu
from jax.experimental.pallas import tpu_sc as plsc

_N_CODES = 8192
_D = 32
_TB = 256


def _argmin_block(z_ref, w_ref, idx_ref):
    z = z_ref[...]
    w = w_ref[...]
    n = jnp.sqrt(jnp.sum(z * z, axis=1, keepdims=True))
    zn = z / jnp.maximum(n, 1e-12)
    z2 = jnp.sum(zn * zn, axis=1, keepdims=True)
    w2 = jnp.sum(w * w, axis=1)[None, :]
    zb = zn.astype(jnp.bfloat16)
    wb = w.astype(jnp.bfloat16)
    dot = lax.dot_general(zb, wb, (((1,), (1,)), ((), ())),
                          preferred_element_type=jnp.float32)
    d = (z2 + w2) - 2.0 * dot
    h = _N_CODES // 2
    d1, d2 = d[:, :h], d[:, h:]
    iota = lax.broadcasted_iota(jnp.int32, d1.shape, 1)
    m1 = jnp.min(d1, axis=1, keepdims=True)
    m2 = jnp.min(d2, axis=1, keepdims=True)
    i1 = jnp.min(jnp.where(d1 == m1, iota, _N_CODES), axis=1)
    i2 = jnp.min(jnp.where(d2 == m2, iota, _N_CODES), axis=1) + h
    r1 = m1[:, 0].astype(jnp.bfloat16).astype(jnp.float32)
    idx = jnp.where(m2[:, 0] < r1, i2, i1)
    idx_ref[0, 0, :] = idx


_NC = 2
_NS = 16
_NW = _NC * _NS
_B = 16 * 1024
_BPW = _B // _NW


@functools.cache
def _sc_gather_fn():
    @functools.partial(
        pl.kernel,
        mesh=plsc.VectorSubcoreMesh(core_axis_name="c", subcore_axis_name="s"),
        compiler_params=pltpu.CompilerParams(use_tc_tiling_on_sc=False),
        out_type=jax.ShapeDtypeStruct((_B, _D), jnp.float32),
        scratch_types=[
            pltpu.VMEM((_BPW,), jnp.int32),
            pltpu.VMEM((_BPW, _D), jnp.float32),
            pltpu.SemaphoreType.DMA,
        ],
    )
    def _sc_gather(table_hbm, idx_hbm, out_hbm, idx_v, rows_v, sem):
        wid = lax.axis_index("s") * _NC + lax.axis_index("c")
        base = wid * _BPW
        pltpu.sync_copy(idx_hbm.at[pl.ds(base, _BPW)], idx_v)
        pltpu.async_copy(table_hbm.at[idx_v], rows_v, sem).wait()
        pltpu.sync_copy(rows_v, out_hbm.at[pl.ds(base, _BPW)])

    return _sc_gather


def kernel(z, weight):
    zf = z.reshape(-1, _D)
    g = zf.shape[0] // _TB
    idx3 = pl.pallas_call(
        _argmin_block,
        grid=(g,),
        in_specs=[
            pl.BlockSpec((_TB, _D), lambda i: (i, 0)),
            pl.BlockSpec((_N_CODES, _D), lambda i: (0, 0)),
        ],
        out_specs=pl.BlockSpec((1, 1, _TB), lambda i: (i, 0, 0)),
        out_shape=jax.ShapeDtypeStruct((g, 1, _TB), jnp.int32),
    )(zf, weight)
    idx = idx3.reshape(-1)
    z_q = _sc_gather_fn()(weight, idx)
    return z_q.reshape(z.shape), idx.reshape(z.shape[:-1])


# speedup vs baseline: 1.2500x; 1.2427x over previous
import functools

import jax
import jax.numpy as jnp
from jax import lax
from jax.experimental import pallas as pl
from jax.experimental.pallas import tpu as pltpu
from jax.experimental.pallas import tpu_sc as plsc

_N_CODES = 8192
_D = 32
_TB = 1024


def _argmin_block(z_ref, w_ref, idx_ref):
    z = z_ref[...]
    w = w_ref[...]
    n = jnp.sqrt(jnp.sum(z * z, axis=1, keepdims=True))
    zn = z / jnp.maximum(n, 1e-12)
    z2 = jnp.sum(zn * zn, axis=1, keepdims=True)
    w2 = jnp.sum(w * w, axis=1)[None, :]
    zb = zn.astype(jnp.bfloat16)
    wb = w.astype(jnp.bfloat16)
    dot = lax.dot_general(zb, wb, (((1,), (1,)), ((), ())),
                          preferred_element_type=jnp.float32)
    d = (z2 + w2) - 2.0 * dot
    h = _N_CODES // 2
    d1, d2 = d[:, :h], d[:, h:]
    iota = lax.broadcasted_iota(jnp.int32, d1.shape, 1)
    m1 = jnp.min(d1, axis=1, keepdims=True)
    m2 = jnp.min(d2, axis=1, keepdims=True)
    i1 = jnp.min(jnp.where(d1 == m1, iota, _N_CODES), axis=1)
    i2 = jnp.min(jnp.where(d2 == m2, iota, _N_CODES), axis=1) + h
    r1 = m1[:, 0].astype(jnp.bfloat16).astype(jnp.float32)
    idx = jnp.where(m2[:, 0] < r1, i2, i1)
    idx_ref[0, 0, :] = idx


_NC = 2
_NS = 16
_NW = _NC * _NS
_B = 16 * 1024
_BPW = _B // _NW


@functools.cache
def _sc_gather_fn():
    @functools.partial(
        pl.kernel,
        mesh=plsc.VectorSubcoreMesh(core_axis_name="c", subcore_axis_name="s"),
        compiler_params=pltpu.CompilerParams(use_tc_tiling_on_sc=False),
        out_type=jax.ShapeDtypeStruct((_B, _D), jnp.float32),
        scratch_types=[
            pltpu.VMEM((_BPW,), jnp.int32),
            pltpu.VMEM((_BPW, _D), jnp.float32),
            pltpu.SemaphoreType.DMA,
        ],
    )
    def _sc_gather(table_hbm, idx_hbm, out_hbm, idx_v, rows_v, sem):
        wid = lax.axis_index("s") * _NC + lax.axis_index("c")
        base = wid * _BPW
        pltpu.sync_copy(idx_hbm.at[pl.ds(base, _BPW)], idx_v)
        pltpu.async_copy(table_hbm.at[idx_v], rows_v, sem).wait()
        pltpu.sync_copy(rows_v, out_hbm.at[pl.ds(base, _BPW)])

    return _sc_gather


def kernel(z, weight):
    zf = z.reshape(-1, _D)
    g = zf.shape[0] // _TB
    idx3 = pl.pallas_call(
        _argmin_block,
        grid=(g,),
        in_specs=[
            pl.BlockSpec((_TB, _D), lambda i: (i, 0)),
            pl.BlockSpec((_N_CODES, _D), lambda i: (0, 0)),
        ],
        out_specs=pl.BlockSpec((1, 1, _TB), lambda i: (i, 0, 0)),
        out_shape=jax.ShapeDtypeStruct((g, 1, _TB), jnp.int32),
    )(zf, weight)
    idx = idx3.reshape(-1)
    z_q = _sc_gather_fn()(weight, idx)
    return z_q.reshape(z.shape), idx.reshape(z.shape[:-1])
